# baseline (speedup 1.0000x reference)
import functools

import jax
import jax.numpy as jnp
from jax import lax
from jax.experimental import pallas as pl
from jax.experimental.pallas import tpu as pltpu
from jax.experimental.pallas import tpu_sc as plsc


@jax.jit
def _sc_lookup(time, betas, body_pose, global_orient, transl):
    mesh = plsc.ScalarSubcoreMesh(axis_name="core", num_cores=2)

    @functools.partial(
        pl.kernel,
        out_type=(
            jax.ShapeDtypeStruct((1, 10), jnp.float32),
            jax.ShapeDtypeStruct((1, 69), jnp.float32),
            jax.ShapeDtypeStruct((1, 3), jnp.float32),
            jax.ShapeDtypeStruct((1, 3), jnp.float32),
        ),
        mesh=mesh,
        scratch_types=[
            pltpu.SMEM((1,), jnp.int32),
            pltpu.SemaphoreType.DMA,
        ],
    )
    def k(time_ref, betas_ref, bp_ref, go_ref, tr_ref,
          betas_out, bp_out, go_out, tr_out, t_smem, sem):
        @pl.when(lax.axis_index("core") == 0)
        def _():
            c0 = pltpu.async_copy(betas_ref, betas_out, sem)
            pltpu.async_copy(time_ref, t_smem, sem).wait()
            t = t_smem[0]
            c1 = pltpu.async_copy(bp_ref.at[t], bp_out, sem)
            c2 = pltpu.async_copy(go_ref.at[t], go_out, sem)
            c3 = pltpu.async_copy(tr_ref.at[t], tr_out, sem)
            c0.wait()
            c1.wait()
            c2.wait()
            c3.wait()

    return k(time, betas, body_pose, global_orient, transl)


def kernel(time, betas, body_pose, global_orient, transl):
    return _sc_lookup(
        time.astype(jnp.int32), betas, body_pose, global_orient, transl
    )


# speedup vs baseline: 3.6393x; 3.6393x over previous
import jax
import jax.numpy as jnp
from jax.experimental import pallas as pl
from jax.experimental.pallas import tpu as pltpu


def _body(t_ref, betas_ref, bp_ref, go_ref, tr_ref,
          b_out, bp_out, go_out, tr_out):
    b_out[...] = betas_ref[...]
    bp_out[...] = bp_ref[0]
    go_out[...] = go_ref[0]
    tr_out[...] = tr_ref[0]


@jax.jit
def _tc_lookup(time, betas, body_pose, global_orient, transl):
    grid_spec = pltpu.PrefetchScalarGridSpec(
        num_scalar_prefetch=1,
        grid=(1,),
        in_specs=[
            pl.BlockSpec((1, 10), lambda i, t: (0, 0)),
            pl.BlockSpec((1, 1, 69), lambda i, t: (t[0], 0, 0)),
            pl.BlockSpec((1, 1, 3), lambda i, t: (t[0], 0, 0)),
            pl.BlockSpec((1, 1, 3), lambda i, t: (t[0], 0, 0)),
        ],
        out_specs=[
            pl.BlockSpec((1, 10), lambda i, t: (0, 0)),
            pl.BlockSpec((1, 69), lambda i, t: (0, 0)),
            pl.BlockSpec((1, 3), lambda i, t: (0, 0)),
            pl.BlockSpec((1, 3), lambda i, t: (0, 0)),
        ],
    )
    return pl.pallas_call(
        _body,
        grid_spec=grid_spec,
        out_shape=(
            jax.ShapeDtypeStruct((1, 10), jnp.float32),
            jax.ShapeDtypeStruct((1, 69), jnp.float32),
            jax.ShapeDtypeStruct((1, 3), jnp.float32),
            jax.ShapeDtypeStruct((1, 3), jnp.float32),
        ),
    )(time, betas, body_pose, global_orient, transl)


def kernel(time, betas, body_pose, global_orient, transl):
    return _tc_lookup(
        time.astype(jnp.int32), betas, body_pose, global_orient, transl
    )


# speedup vs baseline: 3.7117x; 1.0199x over previous
import jax
import jax.numpy as jnp
from jax.experimental import pallas as pl
from jax.experimental.pallas import tpu as pltpu


def _body(t_ref, betas_ref, bp_ref, go_ref, tr_ref,
          b_out, bp_out, go_out, tr_out, sem):
    t = t_ref[0]
    c0 = pltpu.make_async_copy(betas_ref, b_out, sem)
    c1 = pltpu.make_async_copy(bp_ref.at[t], bp_out, sem)
    c2 = pltpu.make_async_copy(go_ref.at[t], go_out, sem)
    c3 = pltpu.make_async_copy(tr_ref.at[t], tr_out, sem)
    c0.start()
    c1.start()
    c2.start()
    c3.start()
    c0.wait()
    c1.wait()
    c2.wait()
    c3.wait()


@jax.jit
def _tc_lookup(time, betas, body_pose, global_orient, transl):
    return pl.pallas_call(
        _body,
        in_specs=[
            pl.BlockSpec(memory_space=pltpu.MemorySpace.SMEM),
            pl.BlockSpec(memory_space=pl.ANY),
            pl.BlockSpec(memory_space=pl.ANY),
            pl.BlockSpec(memory_space=pl.ANY),
            pl.BlockSpec(memory_space=pl.ANY),
        ],
        out_specs=[
            pl.BlockSpec(memory_space=pl.ANY),
            pl.BlockSpec(memory_space=pl.ANY),
            pl.BlockSpec(memory_space=pl.ANY),
            pl.BlockSpec(memory_space=pl.ANY),
        ],
        out_shape=(
            jax.ShapeDtypeStruct((1, 10), jnp.float32),
            jax.ShapeDtypeStruct((1, 69), jnp.float32),
            jax.ShapeDtypeStruct((1, 3), jnp.float32),
            jax.ShapeDtypeStruct((1, 3), jnp.float32),
        ),
        scratch_shapes=[pltpu.SemaphoreType.DMA],
    )(time, betas, body_pose, global_orient, transl)


def kernel(time, betas, body_pose, global_orient, transl):
    return _tc_lookup(
        time.astype(jnp.int32), betas, body_pose, global_orient, transl
    )
